# Initial kernel scaffold; baseline (speedup 1.0000x reference)
import jax
import jax.numpy as jnp
from jax.experimental import pallas as pl


def kernel(x_m, x_d, mm_data, dd_data, mm_edge_index, dd_edge_index, W_gx1, b_gx1, g_nx1, be_nx1, W_gx2, b_gx2, g_nx2, be_nx2, W_gy1, b_gy1, g_ny1, be_ny1, W_gy2, b_gy2, g_ny2, be_ny2, W_lx1, b_lx1, W_lx2, b_lx2, W_lx3, b_lx3, W_ly1, b_ly1, W_ly2, b_ly2, W_ly3, b_ly3):
    raise NotImplementedError("write your pallas kernel here")


# speedup vs baseline: 9.3128x; 9.3128x over previous
import functools

import jax
import jax.numpy as jnp
from jax import lax
from jax.experimental import pallas as pl
from jax.experimental.pallas import tpu as pltpu
from jax.experimental.pallas import tpu_sc as plsc

N = 8192
E = 262144
F = 128
NC, NS = 2, 16
NW = NC * NS
EPW = E // NW
RPS = N // NS

_MESH = dict(core_axis_name="c", subcore_axis_name="s")


def _sc_prep(data_flat, s_idx, t_idx):
    CH = 128
    NCHUNK = EPW // CH

    @functools.partial(
        pl.kernel,
        mesh=plsc.VectorSubcoreMesh(**_MESH),
        out_type=[
            jax.ShapeDtypeStruct((E,), jnp.float32),
            jax.ShapeDtypeStruct((NC, N), jnp.float32),
        ],
        scratch_types=[
            pltpu.VMEM((EPW,), jnp.int32),
            pltpu.VMEM((EPW,), jnp.int32),
            pltpu.VMEM((CH,), jnp.int32),
            pltpu.VMEM((CH,), jnp.int32),
            pltpu.VMEM((CH,), jnp.float32),
            pltpu.VMEM((RPS,), jnp.float32),
            pltpu.VMEM_SHARED((N,), jnp.float32),
            pltpu.SemaphoreType.DMA,
        ],
    )
    def kfn(data_hbm, s_hbm, t_hbm, ea_out, deg_out,
            s_v, t_v, gidx, tidx, eab, zb, deg_sh, sem):
        cid = lax.axis_index("c")
        sid = lax.axis_index("s")
        wid = cid * NS + sid
        base = wid * EPW
        pltpu.sync_copy(s_hbm.at[pl.ds(base, EPW)], s_v)
        pltpu.sync_copy(t_hbm.at[pl.ds(base, EPW)], t_v)

        z16 = jnp.zeros((16,), jnp.float32)

        def zloop(i, carry):
            zb[pl.ds(i * 16, 16)] = z16
            return carry

        lax.fori_loop(0, RPS // 16, zloop, 0)
        pltpu.sync_copy(zb, deg_sh.at[pl.ds(sid * RPS, RPS)])
        plsc.subcore_barrier()

        def chunk(ci, carry):
            cb = ci * CH
            for g in range(CH // 16):
                s16 = s_v[pl.ds(cb + g * 16, 16)]
                t16 = t_v[pl.ds(cb + g * 16, 16)]
                gidx[pl.ds(g * 16, 16)] = s16 * N + t16
                tidx[pl.ds(g * 16, 16)] = t16
            pltpu.async_copy(data_hbm.at[gidx], eab, sem).wait()
            pltpu.sync_copy(eab, ea_out.at[pl.ds(base + cb, CH)])
            pltpu.sync_copy(eab, deg_sh.at[tidx], add=True)
            return carry

        lax.fori_loop(0, NCHUNK, chunk, 0)
        plsc.subcore_barrier()
        pltpu.sync_copy(deg_sh.at[pl.ds(sid * RPS, RPS)],
                        deg_out.at[cid, pl.ds(sid * RPS, RPS)])

    return kfn(data_flat, s_idx, t_idx)


def _sc_agg(xp, s_idx, t_idx, ea):
    CH = 32
    NCHUNK = EPW // CH
    ZR = 64

    @functools.partial(
        pl.kernel,
        mesh=plsc.VectorSubcoreMesh(**_MESH),
        out_type=jax.ShapeDtypeStruct((NC, N, F), jnp.float32),
        scratch_types=[
            pltpu.VMEM((EPW,), jnp.int32),
            pltpu.VMEM((EPW,), jnp.int32),
            pltpu.VMEM((EPW,), jnp.float32),
            pltpu.VMEM((CH,), jnp.int32),
            pltpu.VMEM((CH,), jnp.int32),
            pltpu.VMEM((CH,), jnp.float32),
            pltpu.VMEM((CH, F), jnp.float32),
            pltpu.VMEM((ZR, F), jnp.float32),
            pltpu.VMEM_SHARED((N, F), jnp.float32),
            pltpu.SemaphoreType.DMA,
        ],
    )
    def kfn(xp_hbm, s_hbm, t_hbm, ea_hbm, acc_out,
            s_v, t_v, ea_v, gidx, tidx, eab, rows, zb, acc_sh, sem):
        cid = lax.axis_index("c")
        sid = lax.axis_index("s")
        wid = cid * NS + sid
        base = wid * EPW
        pltpu.sync_copy(s_hbm.at[pl.ds(base, EPW)], s_v)
        pltpu.sync_copy(t_hbm.at[pl.ds(base, EPW)], t_v)
        pltpu.sync_copy(ea_hbm.at[pl.ds(base, EPW)], ea_v)

        z16 = jnp.zeros((16,), jnp.float32)

        def zloop(i, carry):
            r = i // (F // 16)
            q = i % (F // 16)
            zb[r, pl.ds(q * 16, 16)] = z16
            return carry

        lax.fori_loop(0, ZR * (F // 16), zloop, 0)
        for q in range(RPS // ZR):
            pltpu.sync_copy(zb, acc_sh.at[pl.ds(sid * RPS + q * ZR, ZR), :])
        plsc.subcore_barrier()

        def chunk(ci, carry):
            cb = ci * CH
            for g in range(CH // 16):
                s16 = s_v[pl.ds(cb + g * 16, 16)]
                t16 = t_v[pl.ds(cb + g * 16, 16)]
                e16 = ea_v[pl.ds(cb + g * 16, 16)]
                gidx[pl.ds(g * 16, 16)] = s16
                tidx[pl.ds(g * 16, 16)] = t16
                eab[pl.ds(g * 16, 16)] = e16
            pltpu.async_copy(xp_hbm.at[gidx], rows, sem).wait()
            for g in range(CH // 16):
                av = eab[pl.ds(g * 16, 16)]
                for l in range(16):
                    e = g * 16 + l
                    a = av[l]
                    for j in range(F // 16):
                        rows[e, pl.ds(j * 16, 16)] = (
                            rows[e, pl.ds(j * 16, 16)] * a)
            pltpu.sync_copy(rows, acc_sh.at[tidx], add=True)
            return carry

        lax.fori_loop(0, NCHUNK, chunk, 0)
        plsc.subcore_barrier()
        for q in range(RPS // ZR):
            rb = sid * RPS + q * ZR
            pltpu.sync_copy(acc_sh.at[pl.ds(rb, ZR), :],
                            acc_out.at[cid, pl.ds(rb, ZR), :])

    return kfn(xp, s_idx, t_idx, ea)


_BR = 1024


def _dis_block(degp):
    deg = degp[0] + degp[1] + 1.0
    return jnp.where(deg > 0, lax.rsqrt(deg), 0.0)


def _tc_prescale(degp, x):
    def body(degp_ref, x_ref, o_ref):
        dis = _dis_block(degp_ref[...])
        o_ref[...] = x_ref[...] * dis[:, None]

    return pl.pallas_call(
        body,
        grid=(N // _BR,),
        in_specs=[
            pl.BlockSpec((NC, _BR), lambda i: (0, i)),
            pl.BlockSpec((_BR, F), lambda i: (i, 0)),
        ],
        out_specs=pl.BlockSpec((_BR, F), lambda i: (i, 0)),
        out_shape=jax.ShapeDtypeStruct((N, F), jnp.float32),
    )(degp, x)


def _ln(x, g, b):
    mu = jnp.mean(x, axis=-1, keepdims=True)
    var = jnp.mean((x - mu) ** 2, axis=-1, keepdims=True)
    return (x - mu) / jnp.sqrt(var + 1e-5) * g + b


def _tc_mid(degp, acc, xp, W1, b1, g1, be1, W2):
    def body(degp_ref, a_ref, xp_ref, w1_ref, b1_ref, g1_ref, be1_ref,
             w2_ref, o_ref):
        dis = _dis_block(degp_ref[...])[:, None]
        pre = dis * (a_ref[0] + a_ref[1] + xp_ref[...])
        h = jnp.dot(pre, w1_ref[...], preferred_element_type=jnp.float32)
        h = jnp.maximum(h + b1_ref[0], 0.0)
        h = _ln(h, g1_ref[0], be1_ref[0])
        z = jnp.dot(h, w2_ref[...], preferred_element_type=jnp.float32)
        o_ref[...] = dis * z

    C2 = W1.shape[1]
    return pl.pallas_call(
        body,
        grid=(N // _BR,),
        in_specs=[
            pl.BlockSpec((NC, _BR), lambda i: (0, i)),
            pl.BlockSpec((NC, _BR, F), lambda i: (0, i, 0)),
            pl.BlockSpec((_BR, F), lambda i: (i, 0)),
            pl.BlockSpec((F, C2), lambda i: (0, 0)),
            pl.BlockSpec((1, C2), lambda i: (0, 0)),
            pl.BlockSpec((1, C2), lambda i: (0, 0)),
            pl.BlockSpec((1, C2), lambda i: (0, 0)),
            pl.BlockSpec((C2, F), lambda i: (0, 0)),
        ],
        out_specs=pl.BlockSpec((_BR, F), lambda i: (i, 0)),
        out_shape=jax.ShapeDtypeStruct((N, F), jnp.float32),
    )(degp, acc, xp, W1, b1.reshape(1, -1), g1.reshape(1, -1),
      be1.reshape(1, -1), W2)


def _tc_post(degp, acc, zp, b2, g2, be2, Wl1, bl1, Wl2, bl2, Wl3, bl3):
    K = Wl3.shape[1]

    def body(degp_ref, a_ref, zp_ref, b2_ref, g2_ref, be2_ref,
             w1_ref, c1_ref, w2_ref, c2_ref, w3_ref, c3_ref, o_ref):
        dis = _dis_block(degp_ref[...])[:, None]
        x = dis * (a_ref[0] + a_ref[1] + zp_ref[...])
        x = jnp.maximum(x + b2_ref[0], 0.0)
        x = _ln(x, g2_ref[0], be2_ref[0])
        h = jnp.dot(x, w1_ref[...], preferred_element_type=jnp.float32)
        h = jnp.maximum(h + c1_ref[0], 0.0)
        h = jnp.dot(h, w2_ref[...], preferred_element_type=jnp.float32)
        h = jnp.maximum(h + c2_ref[0], 0.0)
        h = jnp.dot(h, w3_ref[...], preferred_element_type=jnp.float32)
        o_ref[...] = jnp.maximum(h + c3_ref[0], 0.0)

    H1, H2 = Wl1.shape[1], Wl2.shape[1]
    return pl.pallas_call(
        body,
        grid=(N // _BR,),
        in_specs=[
            pl.BlockSpec((NC, _BR), lambda i: (0, i)),
            pl.BlockSpec((NC, _BR, F), lambda i: (0, i, 0)),
            pl.BlockSpec((_BR, F), lambda i: (i, 0)),
            pl.BlockSpec((1, F), lambda i: (0, 0)),
            pl.BlockSpec((1, F), lambda i: (0, 0)),
            pl.BlockSpec((1, F), lambda i: (0, 0)),
            pl.BlockSpec((F, H1), lambda i: (0, 0)),
            pl.BlockSpec((1, H1), lambda i: (0, 0)),
            pl.BlockSpec((H1, H2), lambda i: (0, 0)),
            pl.BlockSpec((1, H2), lambda i: (0, 0)),
            pl.BlockSpec((H2, K), lambda i: (0, 0)),
            pl.BlockSpec((1, K), lambda i: (0, 0)),
        ],
        out_specs=pl.BlockSpec((_BR, K), lambda i: (i, 0)),
        out_shape=jax.ShapeDtypeStruct((N, K), jnp.float32),
    )(degp, acc, zp, b2.reshape(1, -1), g2.reshape(1, -1),
      be2.reshape(1, -1), Wl1, bl1.reshape(1, -1), Wl2, bl2.reshape(1, -1),
      Wl3, bl3.reshape(1, -1))


def _tc_score(fx, fy):
    BI, BJ = 1024, 2048
    K = fx.shape[1]

    def body(fx_ref, fy_ref, o_ref):
        o_ref[...] = lax.dot_general(
            fx_ref[...], fy_ref[...], (((1,), (1,)), ((), ())),
            preferred_element_type=jnp.float32)

    return pl.pallas_call(
        body,
        grid=(N // BI, N // BJ),
        in_specs=[
            pl.BlockSpec((BI, K), lambda i, j: (i, 0)),
            pl.BlockSpec((BJ, K), lambda i, j: (j, 0)),
        ],
        out_specs=pl.BlockSpec((BI, BJ), lambda i, j: (i, j)),
        out_shape=jax.ShapeDtypeStruct((N, N), jnp.float32),
    )(fx, fy)


def _branch(x, data, edge_index, W1, b1, g1, be1, W2, b2, g2, be2,
            Wl1, bl1, Wl2, bl2, Wl3, bl3):
    s = edge_index[0].astype(jnp.int32)
    t = edge_index[1].astype(jnp.int32)
    ea, degp = _sc_prep(data.reshape(-1), s, t)
    xp = _tc_prescale(degp, x)
    acc1 = _sc_agg(xp, s, t, ea)
    zp = _tc_mid(degp, acc1, xp, W1, b1, g1, be1, W2)
    acc2 = _sc_agg(zp, s, t, ea)
    return _tc_post(degp, acc2, zp, b2, g2, be2, Wl1, bl1, Wl2, bl2, Wl3, bl3)


def kernel(x_m, x_d, mm_data, dd_data, mm_edge_index, dd_edge_index,
           W_gx1, b_gx1, g_nx1, be_nx1, W_gx2, b_gx2, g_nx2, be_nx2,
           W_gy1, b_gy1, g_ny1, be_ny1, W_gy2, b_gy2, g_ny2, be_ny2,
           W_lx1, b_lx1, W_lx2, b_lx2, W_lx3, b_lx3,
           W_ly1, b_ly1, W_ly2, b_ly2, W_ly3, b_ly3):
    fx = _branch(x_m, mm_data, mm_edge_index,
                 W_gx1, b_gx1, g_nx1, be_nx1, W_gx2, b_gx2, g_nx2, be_nx2,
                 W_lx1, b_lx1, W_lx2, b_lx2, W_lx3, b_lx3)
    fy = _branch(x_d, dd_data, dd_edge_index,
                 W_gy1, b_gy1, g_ny1, be_ny1, W_gy2, b_gy2, g_ny2, be_ny2,
                 W_ly1, b_ly1, W_ly2, b_ly2, W_ly3, b_ly3)
    return _tc_score(fx, fy)


# speedup vs baseline: 14.0104x; 1.5044x over previous
import functools

import jax
import jax.numpy as jnp
from jax import lax
from jax.experimental import pallas as pl
from jax.experimental.pallas import tpu as pltpu
from jax.experimental.pallas import tpu_sc as plsc

N = 8192
E = 262144
F = 128
NC, NS = 2, 16
NW = NC * NS
EPW = E // NW
RPS = N // NS

_MESH = dict(core_axis_name="c", subcore_axis_name="s")


def _sc_prep(data_flat, s_idx, t_idx):
    CH = 128
    NCHUNK = EPW // CH

    @functools.partial(
        pl.kernel,
        mesh=plsc.VectorSubcoreMesh(**_MESH),
        out_type=[
            jax.ShapeDtypeStruct((E,), jnp.float32),
            jax.ShapeDtypeStruct((NC, N), jnp.float32),
        ],
        scratch_types=[
            pltpu.VMEM((EPW,), jnp.int32),
            pltpu.VMEM((EPW,), jnp.int32),
            pltpu.VMEM((CH,), jnp.int32),
            pltpu.VMEM((CH,), jnp.int32),
            pltpu.VMEM((CH,), jnp.int32),
            pltpu.VMEM((CH,), jnp.int32),
            pltpu.VMEM((CH,), jnp.float32),
            pltpu.VMEM((CH,), jnp.float32),
            pltpu.VMEM((RPS,), jnp.float32),
            pltpu.VMEM_SHARED((N,), jnp.float32),
            pltpu.SemaphoreType.DMA,
            pltpu.SemaphoreType.DMA,
            pltpu.SemaphoreType.DMA,
            pltpu.SemaphoreType.DMA,
            pltpu.SemaphoreType.DMA,
            pltpu.SemaphoreType.DMA,
        ],
    )
    def kfn(data_hbm, s_hbm, t_hbm, ea_out, deg_out,
            s_v, t_v, gidx0, gidx1, tidx0, tidx1, eab0, eab1, zb, deg_sh,
            gsem0, gsem1, esem0, esem1, dsem0, dsem1):
        cid = lax.axis_index("c")
        sid = lax.axis_index("s")
        wid = cid * NS + sid
        base = wid * EPW
        slots = ((gidx0, tidx0, eab0, gsem0, esem0, dsem0),
                 (gidx1, tidx1, eab1, gsem1, esem1, dsem1))
        pltpu.sync_copy(s_hbm.at[pl.ds(base, EPW)], s_v)
        pltpu.sync_copy(t_hbm.at[pl.ds(base, EPW)], t_v)

        z16 = jnp.zeros((16,), jnp.float32)

        def zloop(i, carry):
            zb[pl.ds(i * 16, 16)] = z16
            return carry

        lax.fori_loop(0, RPS // 16, zloop, 0)
        pltpu.sync_copy(zb, deg_sh.at[pl.ds(sid * RPS, RPS)])
        plsc.subcore_barrier()

        def build(k, slot):
            gidx, tidx = slot[0], slot[1]
            cb = k * CH
            for g in range(CH // 16):
                s16 = s_v[pl.ds(cb + g * 16, 16)]
                t16 = t_v[pl.ds(cb + g * 16, 16)]
                gidx[pl.ds(g * 16, 16)] = s16 * N + t16
                tidx[pl.ds(g * 16, 16)] = t16

        def fire_gather(slot):
            pltpu.async_copy(data_hbm.at[slot[0]], slot[2], slot[3])

        def wait_gather(slot):
            pltpu.make_async_copy(data_hbm.at[slot[0]], slot[2],
                                  slot[3]).wait()

        def fire_stores(k, slot):
            pltpu.async_copy(slot[2], ea_out.at[pl.ds(base + k * CH, CH)],
                             slot[4])
            pltpu.async_copy(slot[2], deg_sh.at[slot[1]], slot[5], add=True)

        def wait_stores(k, slot):
            pltpu.make_async_copy(slot[2], ea_out.at[pl.ds(base + k * CH, CH)],
                                  slot[4]).wait()
            pltpu.make_async_copy(slot[2], deg_sh.at[slot[1]], slot[5]).wait()

        build(0, slots[0])
        fire_gather(slots[0])

        def body(ci, carry):
            for b in range(2):
                k = ci * 2 + b
                cur, nxt = slots[b], slots[1 - b]

                @pl.when(k + 1 < NCHUNK)
                def _():
                    @pl.when(k > 0)
                    def _():
                        wait_stores(k - 1, nxt)
                    build(k + 1, nxt)
                    fire_gather(nxt)

                wait_gather(cur)
                fire_stores(k, cur)
            return carry

        lax.fori_loop(0, NCHUNK // 2, body, 0)
        wait_stores(NCHUNK - 2, slots[0])
        wait_stores(NCHUNK - 1, slots[1])
        plsc.subcore_barrier()
        pltpu.sync_copy(deg_sh.at[pl.ds(sid * RPS, RPS)],
                        deg_out.at[cid, pl.ds(sid * RPS, RPS)])

    return kfn(data_flat, s_idx, t_idx)


def _sc_agg(xp, s_idx, t_idx, ea):
    CH = 32
    NCHUNK = EPW // CH
    ZR = 64

    @functools.partial(
        pl.kernel,
        mesh=plsc.VectorSubcoreMesh(**_MESH),
        out_type=jax.ShapeDtypeStruct((NC, N, F), jnp.float32),
        scratch_types=[
            pltpu.VMEM((EPW,), jnp.int32),
            pltpu.VMEM((EPW,), jnp.int32),
            pltpu.VMEM((EPW,), jnp.float32),
            pltpu.VMEM((CH,), jnp.int32),
            pltpu.VMEM((CH,), jnp.int32),
            pltpu.VMEM((CH,), jnp.int32),
            pltpu.VMEM((CH,), jnp.int32),
            pltpu.VMEM((CH,), jnp.float32),
            pltpu.VMEM((CH,), jnp.float32),
            pltpu.VMEM((CH, F), jnp.float32),
            pltpu.VMEM((CH, F), jnp.float32),
            pltpu.VMEM((ZR, F), jnp.float32),
            pltpu.VMEM_SHARED((N, F), jnp.float32),
            pltpu.SemaphoreType.DMA,
            pltpu.SemaphoreType.DMA,
            pltpu.SemaphoreType.DMA,
            pltpu.SemaphoreType.DMA,
        ],
    )
    def kfn(xp_hbm, s_hbm, t_hbm, ea_hbm, acc_out,
            s_v, t_v, ea_v, gidx0, gidx1, tidx0, tidx1, eab0, eab1,
            rows0, rows1, zb, acc_sh, gsem0, gsem1, ssem0, ssem1):
        cid = lax.axis_index("c")
        sid = lax.axis_index("s")
        wid = cid * NS + sid
        base = wid * EPW
        slots = ((gidx0, tidx0, eab0, rows0, gsem0, ssem0),
                 (gidx1, tidx1, eab1, rows1, gsem1, ssem1))
        pltpu.sync_copy(s_hbm.at[pl.ds(base, EPW)], s_v)
        pltpu.sync_copy(t_hbm.at[pl.ds(base, EPW)], t_v)
        pltpu.sync_copy(ea_hbm.at[pl.ds(base, EPW)], ea_v)

        z16 = jnp.zeros((16,), jnp.float32)

        def zloop(i, carry):
            r = i // (F // 16)
            q = i % (F // 16)
            zb[r, pl.ds(q * 16, 16)] = z16
            return carry

        lax.fori_loop(0, ZR * (F // 16), zloop, 0)
        for q in range(RPS // ZR):
            pltpu.sync_copy(zb, acc_sh.at[pl.ds(sid * RPS + q * ZR, ZR), :])
        plsc.subcore_barrier()

        def build(k, slot):
            gidx, tidx, eab = slot[0], slot[1], slot[2]
            cb = k * CH
            for g in range(CH // 16):
                s16 = s_v[pl.ds(cb + g * 16, 16)]
                t16 = t_v[pl.ds(cb + g * 16, 16)]
                e16 = ea_v[pl.ds(cb + g * 16, 16)]
                gidx[pl.ds(g * 16, 16)] = s16
                tidx[pl.ds(g * 16, 16)] = t16
                eab[pl.ds(g * 16, 16)] = e16

        def fire_gather(slot):
            pltpu.async_copy(xp_hbm.at[slot[0]], slot[3], slot[4])

        def wait_gather(slot):
            pltpu.make_async_copy(xp_hbm.at[slot[0]], slot[3], slot[4]).wait()

        def scale(slot):
            eab, rows = slot[2], slot[3]
            for g in range(CH // 16):
                av = eab[pl.ds(g * 16, 16)]
                for l in range(16):
                    e = g * 16 + l
                    a = av[l]
                    for j in range(F // 16):
                        rows[e, pl.ds(j * 16, 16)] = (
                            rows[e, pl.ds(j * 16, 16)] * a)

        def fire_scatter(slot):
            pltpu.async_copy(slot[3], acc_sh.at[slot[1]], slot[5], add=True)

        def wait_scatter(slot):
            pltpu.make_async_copy(slot[3], acc_sh.at[slot[1]], slot[5]).wait()

        build(0, slots[0])
        fire_gather(slots[0])

        def body(ci, carry):
            for b in range(2):
                k = ci * 2 + b
                cur, nxt = slots[b], slots[1 - b]

                @pl.when(k + 1 < NCHUNK)
                def _():
                    @pl.when(k > 0)
                    def _():
                        wait_scatter(nxt)
                    build(k + 1, nxt)
                    fire_gather(nxt)

                wait_gather(cur)
                scale(cur)
                fire_scatter(cur)
            return carry

        lax.fori_loop(0, NCHUNK // 2, body, 0)
        wait_scatter(slots[0])
        wait_scatter(slots[1])
        plsc.subcore_barrier()
        for q in range(RPS // ZR):
            rb = sid * RPS + q * ZR
            pltpu.sync_copy(acc_sh.at[pl.ds(rb, ZR), :],
                            acc_out.at[cid, pl.ds(rb, ZR), :])

    return kfn(xp, s_idx, t_idx, ea)


_BR = 1024


def _dis_block(degp):
    deg = degp[0] + degp[1] + 1.0
    return jnp.where(deg > 0, lax.rsqrt(deg), 0.0)


def _tc_prescale(degp, x):
    def body(degp_ref, x_ref, o_ref):
        dis = _dis_block(degp_ref[...])
        o_ref[...] = x_ref[...] * dis[:, None]

    return pl.pallas_call(
        body,
        grid=(N // _BR,),
        in_specs=[
            pl.BlockSpec((NC, _BR), lambda i: (0, i)),
            pl.BlockSpec((_BR, F), lambda i: (i, 0)),
        ],
        out_specs=pl.BlockSpec((_BR, F), lambda i: (i, 0)),
        out_shape=jax.ShapeDtypeStruct((N, F), jnp.float32),
    )(degp, x)


def _ln(x, g, b):
    mu = jnp.mean(x, axis=-1, keepdims=True)
    var = jnp.mean((x - mu) ** 2, axis=-1, keepdims=True)
    return (x - mu) / jnp.sqrt(var + 1e-5) * g + b


def _tc_mid(degp, acc, xp, W1, b1, g1, be1, W2):
    def body(degp_ref, a_ref, xp_ref, w1_ref, b1_ref, g1_ref, be1_ref,
             w2_ref, o_ref):
        dis = _dis_block(degp_ref[...])[:, None]
        pre = dis * (a_ref[0] + a_ref[1] + xp_ref[...])
        h = jnp.dot(pre, w1_ref[...], preferred_element_type=jnp.float32)
        h = jnp.maximum(h + b1_ref[0], 0.0)
        h = _ln(h, g1_ref[0], be1_ref[0])
        z = jnp.dot(h, w2_ref[...], preferred_element_type=jnp.float32)
        o_ref[...] = dis * z

    C2 = W1.shape[1]
    return pl.pallas_call(
        body,
        grid=(N // _BR,),
        in_specs=[
            pl.BlockSpec((NC, _BR), lambda i: (0, i)),
            pl.BlockSpec((NC, _BR, F), lambda i: (0, i, 0)),
            pl.BlockSpec((_BR, F), lambda i: (i, 0)),
            pl.BlockSpec((F, C2), lambda i: (0, 0)),
            pl.BlockSpec((1, C2), lambda i: (0, 0)),
            pl.BlockSpec((1, C2), lambda i: (0, 0)),
            pl.BlockSpec((1, C2), lambda i: (0, 0)),
            pl.BlockSpec((C2, F), lambda i: (0, 0)),
        ],
        out_specs=pl.BlockSpec((_BR, F), lambda i: (i, 0)),
        out_shape=jax.ShapeDtypeStruct((N, F), jnp.float32),
    )(degp, acc, xp, W1, b1.reshape(1, -1), g1.reshape(1, -1),
      be1.reshape(1, -1), W2)


def _tc_post(degp, acc, zp, b2, g2, be2, Wl1, bl1, Wl2, bl2, Wl3, bl3):
    K = Wl3.shape[1]

    def body(degp_ref, a_ref, zp_ref, b2_ref, g2_ref, be2_ref,
             w1_ref, c1_ref, w2_ref, c2_ref, w3_ref, c3_ref, o_ref):
        dis = _dis_block(degp_ref[...])[:, None]
        x = dis * (a_ref[0] + a_ref[1] + zp_ref[...])
        x = jnp.maximum(x + b2_ref[0], 0.0)
        x = _ln(x, g2_ref[0], be2_ref[0])
        h = jnp.dot(x, w1_ref[...], preferred_element_type=jnp.float32)
        h = jnp.maximum(h + c1_ref[0], 0.0)
        h = jnp.dot(h, w2_ref[...], preferred_element_type=jnp.float32)
        h = jnp.maximum(h + c2_ref[0], 0.0)
        h = jnp.dot(h, w3_ref[...], preferred_element_type=jnp.float32)
        o_ref[...] = jnp.maximum(h + c3_ref[0], 0.0)

    H1, H2 = Wl1.shape[1], Wl2.shape[1]
    return pl.pallas_call(
        body,
        grid=(N // _BR,),
        in_specs=[
            pl.BlockSpec((NC, _BR), lambda i: (0, i)),
            pl.BlockSpec((NC, _BR, F), lambda i: (0, i, 0)),
            pl.BlockSpec((_BR, F), lambda i: (i, 0)),
            pl.BlockSpec((1, F), lambda i: (0, 0)),
            pl.BlockSpec((1, F), lambda i: (0, 0)),
            pl.BlockSpec((1, F), lambda i: (0, 0)),
            pl.BlockSpec((F, H1), lambda i: (0, 0)),
            pl.BlockSpec((1, H1), lambda i: (0, 0)),
            pl.BlockSpec((H1, H2), lambda i: (0, 0)),
            pl.BlockSpec((1, H2), lambda i: (0, 0)),
            pl.BlockSpec((H2, K), lambda i: (0, 0)),
            pl.BlockSpec((1, K), lambda i: (0, 0)),
        ],
        out_specs=pl.BlockSpec((_BR, K), lambda i: (i, 0)),
        out_shape=jax.ShapeDtypeStruct((N, K), jnp.float32),
    )(degp, acc, zp, b2.reshape(1, -1), g2.reshape(1, -1),
      be2.reshape(1, -1), Wl1, bl1.reshape(1, -1), Wl2, bl2.reshape(1, -1),
      Wl3, bl3.reshape(1, -1))


def _tc_score(fx, fy):
    BI, BJ = 1024, 2048
    K = fx.shape[1]

    def body(fx_ref, fy_ref, o_ref):
        o_ref[...] = lax.dot_general(
            fx_ref[...], fy_ref[...], (((1,), (1,)), ((), ())),
            preferred_element_type=jnp.float32)

    return pl.pallas_call(
        body,
        grid=(N // BI, N // BJ),
        in_specs=[
            pl.BlockSpec((BI, K), lambda i, j: (i, 0)),
            pl.BlockSpec((BJ, K), lambda i, j: (j, 0)),
        ],
        out_specs=pl.BlockSpec((BI, BJ), lambda i, j: (i, j)),
        out_shape=jax.ShapeDtypeStruct((N, N), jnp.float32),
    )(fx, fy)


def _branch(x, data, edge_index, W1, b1, g1, be1, W2, b2, g2, be2,
            Wl1, bl1, Wl2, bl2, Wl3, bl3):
    s = edge_index[0].astype(jnp.int32)
    t = edge_index[1].astype(jnp.int32)
    ea, degp = _sc_prep(data.reshape(-1), s, t)
    xp = _tc_prescale(degp, x)
    acc1 = _sc_agg(xp, s, t, ea)
    zp = _tc_mid(degp, acc1, xp, W1, b1, g1, be1, W2)
    acc2 = _sc_agg(zp, s, t, ea)
    return _tc_post(degp, acc2, zp, b2, g2, be2, Wl1, bl1, Wl2, bl2, Wl3, bl3)


def kernel(x_m, x_d, mm_data, dd_data, mm_edge_index, dd_edge_index,
           W_gx1, b_gx1, g_nx1, be_nx1, W_gx2, b_gx2, g_nx2, be_nx2,
           W_gy1, b_gy1, g_ny1, be_ny1, W_gy2, b_gy2, g_ny2, be_ny2,
           W_lx1, b_lx1, W_lx2, b_lx2, W_lx3, b_lx3,
           W_ly1, b_ly1, W_ly2, b_ly2, W_ly3, b_ly3):
    fx = _branch(x_m, mm_data, mm_edge_index,
                 W_gx1, b_gx1, g_nx1, be_nx1, W_gx2, b_gx2, g_nx2, be_nx2,
                 W_lx1, b_lx1, W_lx2, b_lx2, W_lx3, b_lx3)
    fy = _branch(x_d, dd_data, dd_edge_index,
                 W_gy1, b_gy1, g_ny1, be_ny1, W_gy2, b_gy2, g_ny2, be_ny2,
                 W_ly1, b_ly1, W_ly2, b_ly2, W_ly3, b_ly3)
    return _tc_score(fx, fy)


# speedup vs baseline: 14.7148x; 1.0503x over previous
import functools

import jax
import jax.numpy as jnp
from jax import lax
from jax.experimental import pallas as pl
from jax.experimental.pallas import tpu as pltpu
from jax.experimental.pallas import tpu_sc as plsc

N = 8192
E = 262144
F = 128
NC, NS = 2, 16
NW = NC * NS
EPW = E // NW
RPS = N // NS

_MESH = dict(core_axis_name="c", subcore_axis_name="s")


def _sc_prep(data_flat, s_idx, t_idx):
    CH = 128
    NCHUNK = EPW // CH

    @functools.partial(
        pl.kernel,
        mesh=plsc.VectorSubcoreMesh(**_MESH),
        out_type=[
            jax.ShapeDtypeStruct((E,), jnp.float32),
            jax.ShapeDtypeStruct((NC, N), jnp.float32),
        ],
        scratch_types=[
            pltpu.VMEM((EPW,), jnp.int32),
            pltpu.VMEM((EPW,), jnp.int32),
            pltpu.VMEM((CH,), jnp.int32),
            pltpu.VMEM((CH,), jnp.int32),
            pltpu.VMEM((CH,), jnp.int32),
            pltpu.VMEM((CH,), jnp.int32),
            pltpu.VMEM((CH,), jnp.float32),
            pltpu.VMEM((CH,), jnp.float32),
            pltpu.VMEM((RPS,), jnp.float32),
            pltpu.VMEM_SHARED((N,), jnp.float32),
            pltpu.SemaphoreType.DMA,
            pltpu.SemaphoreType.DMA,
            pltpu.SemaphoreType.DMA,
            pltpu.SemaphoreType.DMA,
            pltpu.SemaphoreType.DMA,
            pltpu.SemaphoreType.DMA,
        ],
    )
    def kfn(data_hbm, s_hbm, t_hbm, ea_out, deg_out,
            s_v, t_v, gidx0, gidx1, tidx0, tidx1, eab0, eab1, zb, deg_sh,
            gsem0, gsem1, esem0, esem1, dsem0, dsem1):
        cid = lax.axis_index("c")
        sid = lax.axis_index("s")
        wid = cid * NS + sid
        base = wid * EPW
        slots = ((gidx0, tidx0, eab0, gsem0, esem0, dsem0),
                 (gidx1, tidx1, eab1, gsem1, esem1, dsem1))
        pltpu.sync_copy(s_hbm.at[pl.ds(base, EPW)], s_v)
        pltpu.sync_copy(t_hbm.at[pl.ds(base, EPW)], t_v)

        z16 = jnp.zeros((16,), jnp.float32)

        def zloop(i, carry):
            zb[pl.ds(i * 16, 16)] = z16
            return carry

        lax.fori_loop(0, RPS // 16, zloop, 0)
        pltpu.sync_copy(zb, deg_sh.at[pl.ds(sid * RPS, RPS)])
        plsc.subcore_barrier()

        def build(k, slot):
            gidx, tidx = slot[0], slot[1]
            cb = k * CH
            for g in range(CH // 16):
                s16 = s_v[pl.ds(cb + g * 16, 16)]
                t16 = t_v[pl.ds(cb + g * 16, 16)]
                gidx[pl.ds(g * 16, 16)] = s16 * N + t16
                tidx[pl.ds(g * 16, 16)] = t16

        def fire_gather(slot):
            pltpu.async_copy(data_hbm.at[slot[0]], slot[2], slot[3])

        def wait_gather(slot):
            pltpu.make_async_copy(data_hbm.at[slot[0]], slot[2],
                                  slot[3]).wait()

        def fire_stores(k, slot):
            pltpu.async_copy(slot[2], ea_out.at[pl.ds(base + k * CH, CH)],
                             slot[4])
            pltpu.async_copy(slot[2], deg_sh.at[slot[1]], slot[5], add=True)

        def wait_stores(k, slot):
            pltpu.make_async_copy(slot[2], ea_out.at[pl.ds(base + k * CH, CH)],
                                  slot[4]).wait()
            pltpu.make_async_copy(slot[2], deg_sh.at[slot[1]], slot[5]).wait()

        build(0, slots[0])
        fire_gather(slots[0])

        def body(ci, carry):
            for b in range(2):
                k = ci * 2 + b
                cur, nxt = slots[b], slots[1 - b]

                @pl.when(k + 1 < NCHUNK)
                def _():
                    @pl.when(k > 0)
                    def _():
                        wait_stores(k - 1, nxt)
                    build(k + 1, nxt)
                    fire_gather(nxt)

                wait_gather(cur)
                fire_stores(k, cur)
            return carry

        lax.fori_loop(0, NCHUNK // 2, body, 0)
        wait_stores(NCHUNK - 2, slots[0])
        wait_stores(NCHUNK - 1, slots[1])
        plsc.subcore_barrier()
        pltpu.sync_copy(deg_sh.at[pl.ds(sid * RPS, RPS)],
                        deg_out.at[cid, pl.ds(sid * RPS, RPS)])

    return kfn(data_flat, s_idx, t_idx)


def _sc_agg(xp, s_idx, t_idx, ea):
    CH = 32
    NCHUNK = EPW // CH
    ZR = 64

    @functools.partial(
        pl.kernel,
        mesh=plsc.VectorSubcoreMesh(**_MESH),
        out_type=jax.ShapeDtypeStruct((NC, N, F), jnp.float32),
        scratch_types=[
            pltpu.VMEM((EPW,), jnp.int32),
            pltpu.VMEM((EPW,), jnp.int32),
            pltpu.VMEM((EPW,), jnp.float32),
            pltpu.VMEM((CH,), jnp.int32),
            pltpu.VMEM((CH,), jnp.int32),
            pltpu.VMEM((CH,), jnp.int32),
            pltpu.VMEM((CH,), jnp.int32),
            pltpu.VMEM((CH,), jnp.float32),
            pltpu.VMEM((CH,), jnp.float32),
            pltpu.VMEM((CH, F), jnp.float32),
            pltpu.VMEM((CH, F), jnp.float32),
            pltpu.VMEM((ZR, F), jnp.float32),
            pltpu.VMEM_SHARED((N, F), jnp.float32),
            pltpu.SemaphoreType.DMA,
            pltpu.SemaphoreType.DMA,
            pltpu.SemaphoreType.DMA,
            pltpu.SemaphoreType.DMA,
        ],
    )
    def kfn(xp_hbm, s_hbm, t_hbm, ea_hbm, acc_out,
            s_v, t_v, ea_v, gidx0, gidx1, tidx0, tidx1, eab0, eab1,
            rows0, rows1, zb, acc_sh, gsem0, gsem1, ssem0, ssem1):
        cid = lax.axis_index("c")
        sid = lax.axis_index("s")
        wid = cid * NS + sid
        base = wid * EPW
        slots = ((gidx0, tidx0, eab0, rows0, gsem0, ssem0),
                 (gidx1, tidx1, eab1, rows1, gsem1, ssem1))
        pltpu.sync_copy(s_hbm.at[pl.ds(base, EPW)], s_v)
        pltpu.sync_copy(t_hbm.at[pl.ds(base, EPW)], t_v)
        pltpu.sync_copy(ea_hbm.at[pl.ds(base, EPW)], ea_v)

        z16 = jnp.zeros((16,), jnp.float32)

        def zloop(i, carry):
            r = i // (F // 16)
            q = i % (F // 16)
            zb[r, pl.ds(q * 16, 16)] = z16
            return carry

        lax.fori_loop(0, ZR * (F // 16), zloop, 0)
        for q in range(RPS // ZR):
            pltpu.sync_copy(zb, acc_sh.at[pl.ds(sid * RPS + q * ZR, ZR), :])
        plsc.subcore_barrier()

        def build(k, slot):
            gidx, tidx, eab = slot[0], slot[1], slot[2]
            cb = k * CH
            for g in range(CH // 16):
                s16 = s_v[pl.ds(cb + g * 16, 16)]
                t16 = t_v[pl.ds(cb + g * 16, 16)]
                e16 = ea_v[pl.ds(cb + g * 16, 16)]
                gidx[pl.ds(g * 16, 16)] = s16
                tidx[pl.ds(g * 16, 16)] = t16
                eab[pl.ds(g * 16, 16)] = e16

        def fire_gather(slot):
            pltpu.async_copy(xp_hbm.at[slot[0]], slot[3], slot[4])

        def wait_gather(slot):
            pltpu.make_async_copy(xp_hbm.at[slot[0]], slot[3], slot[4]).wait()

        def scale(slot):
            eab, rows = slot[2], slot[3]
            for g in range(CH // 16):
                av = eab[pl.ds(g * 16, 16)]
                for l in range(16):
                    e = g * 16 + l
                    a = av[l]
                    for j in range(F // 16):
                        rows[e, pl.ds(j * 16, 16)] = (
                            rows[e, pl.ds(j * 16, 16)] * a)

        def fire_scatter(slot):
            pltpu.async_copy(slot[3], acc_sh.at[slot[1]], slot[5], add=True)

        def wait_scatter(slot):
            pltpu.make_async_copy(slot[3], acc_sh.at[slot[1]], slot[5]).wait()

        build(0, slots[0])
        fire_gather(slots[0])

        def body(ci, carry):
            for b in range(2):
                k = ci * 2 + b
                cur, nxt = slots[b], slots[1 - b]

                @pl.when(k + 1 < NCHUNK)
                def _():
                    @pl.when(k > 0)
                    def _():
                        wait_scatter(nxt)
                    build(k + 1, nxt)
                    fire_gather(nxt)

                wait_gather(cur)
                scale(cur)
                fire_scatter(cur)
            return carry

        lax.fori_loop(0, NCHUNK // 2, body, 0)
        wait_scatter(slots[0])
        wait_scatter(slots[1])
        plsc.subcore_barrier()
        for q in range(RPS // ZR):
            rb = sid * RPS + q * ZR
            pltpu.sync_copy(acc_sh.at[pl.ds(rb, ZR), :],
                            acc_out.at[cid, pl.ds(rb, ZR), :])

    return kfn(xp, s_idx, t_idx, ea)


_BR = 1024


def _tc_relayout(data):
    BR = 128

    def body(x_ref, o_ref):
        o_ref[...] = x_ref[...].reshape(BR * (N // 128), 128)

    return pl.pallas_call(
        body,
        grid=(N // BR,),
        in_specs=[pl.BlockSpec((BR, N), lambda i: (i, 0))],
        out_specs=pl.BlockSpec((BR * (N // 128), 128), lambda i: (i, 0)),
        out_shape=jax.ShapeDtypeStruct((N * N // 128, 128), jnp.float32),
    )(data)


def _dis_block(degp):
    deg = degp[0] + degp[1] + 1.0
    return jnp.where(deg > 0, lax.rsqrt(deg), 0.0)


def _tc_prescale(degp, x):
    def body(degp_ref, x_ref, o_ref):
        dis = _dis_block(degp_ref[...])
        o_ref[...] = x_ref[...] * dis[:, None]

    return pl.pallas_call(
        body,
        grid=(N // _BR,),
        in_specs=[
            pl.BlockSpec((NC, _BR), lambda i: (0, i)),
            pl.BlockSpec((_BR, F), lambda i: (i, 0)),
        ],
        out_specs=pl.BlockSpec((_BR, F), lambda i: (i, 0)),
        out_shape=jax.ShapeDtypeStruct((N, F), jnp.float32),
    )(degp, x)


def _ln(x, g, b):
    mu = jnp.mean(x, axis=-1, keepdims=True)
    var = jnp.mean((x - mu) ** 2, axis=-1, keepdims=True)
    return (x - mu) / jnp.sqrt(var + 1e-5) * g + b


def _tc_mid(degp, acc, xp, W1, b1, g1, be1, W2):
    def body(degp_ref, a_ref, xp_ref, w1_ref, b1_ref, g1_ref, be1_ref,
             w2_ref, o_ref):
        dis = _dis_block(degp_ref[...])[:, None]
        pre = dis * (a_ref[0] + a_ref[1] + xp_ref[...])
        h = jnp.dot(pre, w1_ref[...], preferred_element_type=jnp.float32)
        h = jnp.maximum(h + b1_ref[0], 0.0)
        h = _ln(h, g1_ref[0], be1_ref[0])
        z = jnp.dot(h, w2_ref[...], preferred_element_type=jnp.float32)
        o_ref[...] = dis * z

    C2 = W1.shape[1]
    return pl.pallas_call(
        body,
        grid=(N // _BR,),
        in_specs=[
            pl.BlockSpec((NC, _BR), lambda i: (0, i)),
            pl.BlockSpec((NC, _BR, F), lambda i: (0, i, 0)),
            pl.BlockSpec((_BR, F), lambda i: (i, 0)),
            pl.BlockSpec((F, C2), lambda i: (0, 0)),
            pl.BlockSpec((1, C2), lambda i: (0, 0)),
            pl.BlockSpec((1, C2), lambda i: (0, 0)),
            pl.BlockSpec((1, C2), lambda i: (0, 0)),
            pl.BlockSpec((C2, F), lambda i: (0, 0)),
        ],
        out_specs=pl.BlockSpec((_BR, F), lambda i: (i, 0)),
        out_shape=jax.ShapeDtypeStruct((N, F), jnp.float32),
    )(degp, acc, xp, W1, b1.reshape(1, -1), g1.reshape(1, -1),
      be1.reshape(1, -1), W2)


def _tc_post(degp, acc, zp, b2, g2, be2, Wl1, bl1, Wl2, bl2, Wl3, bl3):
    K = Wl3.shape[1]

    def body(degp_ref, a_ref, zp_ref, b2_ref, g2_ref, be2_ref,
             w1_ref, c1_ref, w2_ref, c2_ref, w3_ref, c3_ref, o_ref):
        dis = _dis_block(degp_ref[...])[:, None]
        x = dis * (a_ref[0] + a_ref[1] + zp_ref[...])
        x = jnp.maximum(x + b2_ref[0], 0.0)
        x = _ln(x, g2_ref[0], be2_ref[0])
        h = jnp.dot(x, w1_ref[...], preferred_element_type=jnp.float32)
        h = jnp.maximum(h + c1_ref[0], 0.0)
        h = jnp.dot(h, w2_ref[...], preferred_element_type=jnp.float32)
        h = jnp.maximum(h + c2_ref[0], 0.0)
        h = jnp.dot(h, w3_ref[...], preferred_element_type=jnp.float32)
        o_ref[...] = jnp.maximum(h + c3_ref[0], 0.0)

    H1, H2 = Wl1.shape[1], Wl2.shape[1]
    return pl.pallas_call(
        body,
        grid=(N // _BR,),
        in_specs=[
            pl.BlockSpec((NC, _BR), lambda i: (0, i)),
            pl.BlockSpec((NC, _BR, F), lambda i: (0, i, 0)),
            pl.BlockSpec((_BR, F), lambda i: (i, 0)),
            pl.BlockSpec((1, F), lambda i: (0, 0)),
            pl.BlockSpec((1, F), lambda i: (0, 0)),
            pl.BlockSpec((1, F), lambda i: (0, 0)),
            pl.BlockSpec((F, H1), lambda i: (0, 0)),
            pl.BlockSpec((1, H1), lambda i: (0, 0)),
            pl.BlockSpec((H1, H2), lambda i: (0, 0)),
            pl.BlockSpec((1, H2), lambda i: (0, 0)),
            pl.BlockSpec((H2, K), lambda i: (0, 0)),
            pl.BlockSpec((1, K), lambda i: (0, 0)),
        ],
        out_specs=pl.BlockSpec((_BR, K), lambda i: (i, 0)),
        out_shape=jax.ShapeDtypeStruct((N, K), jnp.float32),
    )(degp, acc, zp, b2.reshape(1, -1), g2.reshape(1, -1),
      be2.reshape(1, -1), Wl1, bl1.reshape(1, -1), Wl2, bl2.reshape(1, -1),
      Wl3, bl3.reshape(1, -1))


def _tc_score(fx, fy):
    BI, BJ = 1024, 2048
    K = fx.shape[1]

    def body(fx_ref, fy_ref, o_ref):
        o_ref[...] = lax.dot_general(
            fx_ref[...], fy_ref[...], (((1,), (1,)), ((), ())),
            preferred_element_type=jnp.float32)

    return pl.pallas_call(
        body,
        grid=(N // BI, N // BJ),
        in_specs=[
            pl.BlockSpec((BI, K), lambda i, j: (i, 0)),
            pl.BlockSpec((BJ, K), lambda i, j: (j, 0)),
        ],
        out_specs=pl.BlockSpec((BI, BJ), lambda i, j: (i, j)),
        out_shape=jax.ShapeDtypeStruct((N, N), jnp.float32),
    )(fx, fy)


def _branch(x, data, edge_index, W1, b1, g1, be1, W2, b2, g2, be2,
            Wl1, bl1, Wl2, bl2, Wl3, bl3):
    s = edge_index[0].astype(jnp.int32)
    t = edge_index[1].astype(jnp.int32)
    ea, degp = _sc_prep(_tc_relayout(data).reshape(-1), s, t)
    xp = _tc_prescale(degp, x)
    acc1 = _sc_agg(xp, s, t, ea)
    zp = _tc_mid(degp, acc1, xp, W1, b1, g1, be1, W2)
    acc2 = _sc_agg(zp, s, t, ea)
    return _tc_post(degp, acc2, zp, b2, g2, be2, Wl1, bl1, Wl2, bl2, Wl3, bl3)


def kernel(x_m, x_d, mm_data, dd_data, mm_edge_index, dd_edge_index,
           W_gx1, b_gx1, g_nx1, be_nx1, W_gx2, b_gx2, g_nx2, be_nx2,
           W_gy1, b_gy1, g_ny1, be_ny1, W_gy2, b_gy2, g_ny2, be_ny2,
           W_lx1, b_lx1, W_lx2, b_lx2, W_lx3, b_lx3,
           W_ly1, b_ly1, W_ly2, b_ly2, W_ly3, b_ly3):
    fx = _branch(x_m, mm_data, mm_edge_index,
                 W_gx1, b_gx1, g_nx1, be_nx1, W_gx2, b_gx2, g_nx2, be_nx2,
                 W_lx1, b_lx1, W_lx2, b_lx2, W_lx3, b_lx3)
    fy = _branch(x_d, dd_data, dd_edge_index,
                 W_gy1, b_gy1, g_ny1, be_ny1, W_gy2, b_gy2, g_ny2, be_ny2,
                 W_ly1, b_ly1, W_ly2, b_ly2, W_ly3, b_ly3)
    return _tc_score(fx, fy)


# speedup vs baseline: 17.0256x; 1.1570x over previous
import functools

import jax
import jax.numpy as jnp
from jax import lax
from jax.experimental import pallas as pl
from jax.experimental.pallas import tpu as pltpu
from jax.experimental.pallas import tpu_sc as plsc

N = 8192
E = 262144
F = 128
NC, NS = 2, 16
NW = NC * NS
EPW = E // NW
RPS = N // NS

_MESH = dict(core_axis_name="c", subcore_axis_name="s")


def _sc_prep(data_flat, s_idx, t_idx):
    CH = 128
    NCHUNK = EPW // CH

    @functools.partial(
        pl.kernel,
        mesh=plsc.VectorSubcoreMesh(**_MESH),
        out_type=[
            jax.ShapeDtypeStruct((E,), jnp.float32),
            jax.ShapeDtypeStruct((NC, N), jnp.float32),
        ],
        scratch_types=[
            pltpu.VMEM((EPW,), jnp.int32),
            pltpu.VMEM((EPW,), jnp.int32),
            pltpu.VMEM((CH,), jnp.int32),
            pltpu.VMEM((CH,), jnp.int32),
            pltpu.VMEM((CH,), jnp.int32),
            pltpu.VMEM((CH,), jnp.int32),
            pltpu.VMEM((CH,), jnp.float32),
            pltpu.VMEM((CH,), jnp.float32),
            pltpu.VMEM((RPS,), jnp.float32),
            pltpu.VMEM_SHARED((N,), jnp.float32),
            pltpu.SemaphoreType.DMA,
            pltpu.SemaphoreType.DMA,
            pltpu.SemaphoreType.DMA,
            pltpu.SemaphoreType.DMA,
            pltpu.SemaphoreType.DMA,
            pltpu.SemaphoreType.DMA,
        ],
    )
    def kfn(data_hbm, s_hbm, t_hbm, ea_out, deg_out,
            s_v, t_v, gidx0, gidx1, tidx0, tidx1, eab0, eab1, zb, deg_sh,
            gsem0, gsem1, esem0, esem1, dsem0, dsem1):
        cid = lax.axis_index("c")
        sid = lax.axis_index("s")
        wid = cid * NS + sid
        base = wid * EPW
        slots = ((gidx0, tidx0, eab0, gsem0, esem0, dsem0),
                 (gidx1, tidx1, eab1, gsem1, esem1, dsem1))
        pltpu.sync_copy(s_hbm.at[pl.ds(base, EPW)], s_v)
        pltpu.sync_copy(t_hbm.at[pl.ds(base, EPW)], t_v)

        z16 = jnp.zeros((16,), jnp.float32)

        def zloop(i, carry):
            zb[pl.ds(i * 16, 16)] = z16
            return carry

        lax.fori_loop(0, RPS // 16, zloop, 0)
        pltpu.sync_copy(zb, deg_sh.at[pl.ds(sid * RPS, RPS)])
        plsc.subcore_barrier()

        def build(k, slot):
            gidx, tidx = slot[0], slot[1]
            cb = k * CH
            for g in range(CH // 16):
                s16 = s_v[pl.ds(cb + g * 16, 16)]
                t16 = t_v[pl.ds(cb + g * 16, 16)]
                gidx[pl.ds(g * 16, 16)] = s16 * N + t16
                tidx[pl.ds(g * 16, 16)] = t16

        def fire_gather(slot):
            pltpu.async_copy(data_hbm.at[slot[0]], slot[2], slot[3])

        def wait_gather(slot):
            pltpu.make_async_copy(data_hbm.at[slot[0]], slot[2],
                                  slot[3]).wait()

        def fire_stores(k, slot):
            pltpu.async_copy(slot[2], ea_out.at[pl.ds(base + k * CH, CH)],
                             slot[4])
            pltpu.async_copy(slot[2], deg_sh.at[slot[1]], slot[5], add=True)

        def wait_stores(k, slot):
            pltpu.make_async_copy(slot[2], ea_out.at[pl.ds(base + k * CH, CH)],
                                  slot[4]).wait()
            pltpu.make_async_copy(slot[2], deg_sh.at[slot[1]], slot[5]).wait()

        build(0, slots[0])
        fire_gather(slots[0])

        def body(ci, carry):
            for b in range(2):
                k = ci * 2 + b
                cur, nxt = slots[b], slots[1 - b]

                @pl.when(k + 1 < NCHUNK)
                def _():
                    @pl.when(k > 0)
                    def _():
                        wait_stores(k - 1, nxt)
                    build(k + 1, nxt)
                    fire_gather(nxt)

                wait_gather(cur)
                fire_stores(k, cur)
            return carry

        lax.fori_loop(0, NCHUNK // 2, body, 0)
        wait_stores(NCHUNK - 2, slots[0])
        wait_stores(NCHUNK - 1, slots[1])
        plsc.subcore_barrier()
        pltpu.sync_copy(deg_sh.at[pl.ds(sid * RPS, RPS)],
                        deg_out.at[cid, pl.ds(sid * RPS, RPS)])

    return kfn(data_flat, s_idx, t_idx)


def _sc_agg(xp, s_idx, t_idx, ea):
    CH = 64
    NCHUNK = EPW // CH
    ZR = 64

    @functools.partial(
        pl.kernel,
        mesh=plsc.VectorSubcoreMesh(**_MESH),
        out_type=jax.ShapeDtypeStruct((NC, N, F), jnp.float32),
        scratch_types=[
            pltpu.VMEM((EPW,), jnp.int32),
            pltpu.VMEM((EPW,), jnp.int32),
            pltpu.VMEM((EPW,), jnp.float32),
            pltpu.VMEM((CH,), jnp.int32),
            pltpu.VMEM((CH,), jnp.int32),
            pltpu.VMEM((CH,), jnp.int32),
            pltpu.VMEM((CH,), jnp.int32),
            pltpu.VMEM((CH,), jnp.float32),
            pltpu.VMEM((CH,), jnp.float32),
            pltpu.VMEM((CH, F), jnp.float32),
            pltpu.VMEM((CH, F), jnp.float32),
            pltpu.VMEM((ZR, F), jnp.float32),
            pltpu.VMEM_SHARED((N, F), jnp.float32),
            pltpu.SemaphoreType.DMA,
            pltpu.SemaphoreType.DMA,
            pltpu.SemaphoreType.DMA,
            pltpu.SemaphoreType.DMA,
        ],
    )
    def kfn(xp_hbm, s_hbm, t_hbm, ea_hbm, acc_out,
            s_v, t_v, ea_v, gidx0, gidx1, tidx0, tidx1, eab0, eab1,
            rows0, rows1, zb, acc_sh, gsem0, gsem1, ssem0, ssem1):
        cid = lax.axis_index("c")
        sid = lax.axis_index("s")
        wid = cid * NS + sid
        base = wid * EPW
        slots = ((gidx0, tidx0, eab0, rows0, gsem0, ssem0),
                 (gidx1, tidx1, eab1, rows1, gsem1, ssem1))
        pltpu.sync_copy(s_hbm.at[pl.ds(base, EPW)], s_v)
        pltpu.sync_copy(t_hbm.at[pl.ds(base, EPW)], t_v)
        pltpu.sync_copy(ea_hbm.at[pl.ds(base, EPW)], ea_v)

        z16 = jnp.zeros((16,), jnp.float32)

        def zloop(i, carry):
            r = i // (F // 16)
            q = i % (F // 16)
            zb[r, pl.ds(q * 16, 16)] = z16
            return carry

        lax.fori_loop(0, ZR * (F // 16), zloop, 0)
        for q in range(RPS // ZR):
            pltpu.sync_copy(zb, acc_sh.at[pl.ds(sid * RPS + q * ZR, ZR), :])
        plsc.subcore_barrier()

        def build(k, slot):
            gidx, tidx, eab = slot[0], slot[1], slot[2]
            cb = k * CH
            for g in range(CH // 16):
                s16 = s_v[pl.ds(cb + g * 16, 16)]
                t16 = t_v[pl.ds(cb + g * 16, 16)]
                e16 = ea_v[pl.ds(cb + g * 16, 16)]
                gidx[pl.ds(g * 16, 16)] = s16
                tidx[pl.ds(g * 16, 16)] = t16
                eab[pl.ds(g * 16, 16)] = e16

        def fire_gather(slot):
            pltpu.async_copy(xp_hbm.at[slot[0]], slot[3], slot[4])

        def wait_gather(slot):
            pltpu.make_async_copy(xp_hbm.at[slot[0]], slot[3], slot[4]).wait()

        def scale(slot):
            eab, rows = slot[2], slot[3]
            for g in range(CH // 16):
                av = eab[pl.ds(g * 16, 16)]
                for l in range(16):
                    e = g * 16 + l
                    a = av[l]
                    for j in range(F // 16):
                        rows[e, pl.ds(j * 16, 16)] = (
                            rows[e, pl.ds(j * 16, 16)] * a)

        def fire_scatter(slot):
            pltpu.async_copy(slot[3], acc_sh.at[slot[1]], slot[5], add=True)

        def wait_scatter(slot):
            pltpu.make_async_copy(slot[3], acc_sh.at[slot[1]], slot[5]).wait()

        build(0, slots[0])
        fire_gather(slots[0])

        def body(ci, carry):
            for b in range(2):
                k = ci * 2 + b
                cur, nxt = slots[b], slots[1 - b]

                @pl.when(k + 1 < NCHUNK)
                def _():
                    @pl.when(k > 0)
                    def _():
                        wait_scatter(nxt)
                    build(k + 1, nxt)
                    fire_gather(nxt)

                wait_gather(cur)
                scale(cur)
                fire_scatter(cur)
            return carry

        lax.fori_loop(0, NCHUNK // 2, body, 0)
        wait_scatter(slots[0])
        wait_scatter(slots[1])
        plsc.subcore_barrier()
        for q in range(RPS // ZR):
            rb = sid * RPS + q * ZR
            pltpu.sync_copy(acc_sh.at[pl.ds(rb, ZR), :],
                            acc_out.at[cid, pl.ds(rb, ZR), :])

    return kfn(xp, s_idx, t_idx, ea)


_BR = 1024


def _tc_relayout(data):
    BR = 128

    def body(x_ref, o_ref):
        o_ref[...] = x_ref[...].reshape(BR * (N // 128), 128)

    return pl.pallas_call(
        body,
        grid=(N // BR,),
        in_specs=[pl.BlockSpec((BR, N), lambda i: (i, 0))],
        out_specs=pl.BlockSpec((BR * (N // 128), 128), lambda i: (i, 0)),
        out_shape=jax.ShapeDtypeStruct((N * N // 128, 128), jnp.float32),
    )(data)


def _dis_block(degp):
    deg = degp[0] + degp[1] + 1.0
    return jnp.where(deg > 0, lax.rsqrt(deg), 0.0)


def _tc_prescale(degp, x):
    def body(degp_ref, x_ref, o_ref):
        dis = _dis_block(degp_ref[...])
        o_ref[...] = x_ref[...] * dis[:, None]

    return pl.pallas_call(
        body,
        grid=(N // _BR,),
        in_specs=[
            pl.BlockSpec((NC, _BR), lambda i: (0, i)),
            pl.BlockSpec((_BR, F), lambda i: (i, 0)),
        ],
        out_specs=pl.BlockSpec((_BR, F), lambda i: (i, 0)),
        out_shape=jax.ShapeDtypeStruct((N, F), jnp.float32),
    )(degp, x)


def _ln(x, g, b):
    mu = jnp.mean(x, axis=-1, keepdims=True)
    var = jnp.mean((x - mu) ** 2, axis=-1, keepdims=True)
    return (x - mu) / jnp.sqrt(var + 1e-5) * g + b


def _tc_mid(degp, acc, xp, W1, b1, g1, be1, W2):
    def body(degp_ref, a_ref, xp_ref, w1_ref, b1_ref, g1_ref, be1_ref,
             w2_ref, o_ref):
        dis = _dis_block(degp_ref[...])[:, None]
        pre = dis * (a_ref[0] + a_ref[1] + xp_ref[...])
        h = jnp.dot(pre, w1_ref[...], preferred_element_type=jnp.float32)
        h = jnp.maximum(h + b1_ref[0], 0.0)
        h = _ln(h, g1_ref[0], be1_ref[0])
        z = jnp.dot(h, w2_ref[...], preferred_element_type=jnp.float32)
        o_ref[...] = dis * z

    C2 = W1.shape[1]
    return pl.pallas_call(
        body,
        grid=(N // _BR,),
        in_specs=[
            pl.BlockSpec((NC, _BR), lambda i: (0, i)),
            pl.BlockSpec((NC, _BR, F), lambda i: (0, i, 0)),
            pl.BlockSpec((_BR, F), lambda i: (i, 0)),
            pl.BlockSpec((F, C2), lambda i: (0, 0)),
            pl.BlockSpec((1, C2), lambda i: (0, 0)),
            pl.BlockSpec((1, C2), lambda i: (0, 0)),
            pl.BlockSpec((1, C2), lambda i: (0, 0)),
            pl.BlockSpec((C2, F), lambda i: (0, 0)),
        ],
        out_specs=pl.BlockSpec((_BR, F), lambda i: (i, 0)),
        out_shape=jax.ShapeDtypeStruct((N, F), jnp.float32),
    )(degp, acc, xp, W1, b1.reshape(1, -1), g1.reshape(1, -1),
      be1.reshape(1, -1), W2)


def _tc_post(degp, acc, zp, b2, g2, be2, Wl1, bl1, Wl2, bl2, Wl3, bl3):
    K = Wl3.shape[1]

    def body(degp_ref, a_ref, zp_ref, b2_ref, g2_ref, be2_ref,
             w1_ref, c1_ref, w2_ref, c2_ref, w3_ref, c3_ref, o_ref):
        dis = _dis_block(degp_ref[...])[:, None]
        x = dis * (a_ref[0] + a_ref[1] + zp_ref[...])
        x = jnp.maximum(x + b2_ref[0], 0.0)
        x = _ln(x, g2_ref[0], be2_ref[0])
        h = jnp.dot(x, w1_ref[...], preferred_element_type=jnp.float32)
        h = jnp.maximum(h + c1_ref[0], 0.0)
        h = jnp.dot(h, w2_ref[...], preferred_element_type=jnp.float32)
        h = jnp.maximum(h + c2_ref[0], 0.0)
        h = jnp.dot(h, w3_ref[...], preferred_element_type=jnp.float32)
        o_ref[...] = jnp.maximum(h + c3_ref[0], 0.0)

    H1, H2 = Wl1.shape[1], Wl2.shape[1]
    return pl.pallas_call(
        body,
        grid=(N // _BR,),
        in_specs=[
            pl.BlockSpec((NC, _BR), lambda i: (0, i)),
            pl.BlockSpec((NC, _BR, F), lambda i: (0, i, 0)),
            pl.BlockSpec((_BR, F), lambda i: (i, 0)),
            pl.BlockSpec((1, F), lambda i: (0, 0)),
            pl.BlockSpec((1, F), lambda i: (0, 0)),
            pl.BlockSpec((1, F), lambda i: (0, 0)),
            pl.BlockSpec((F, H1), lambda i: (0, 0)),
            pl.BlockSpec((1, H1), lambda i: (0, 0)),
            pl.BlockSpec((H1, H2), lambda i: (0, 0)),
            pl.BlockSpec((1, H2), lambda i: (0, 0)),
            pl.BlockSpec((H2, K), lambda i: (0, 0)),
            pl.BlockSpec((1, K), lambda i: (0, 0)),
        ],
        out_specs=pl.BlockSpec((_BR, K), lambda i: (i, 0)),
        out_shape=jax.ShapeDtypeStruct((N, K), jnp.float32),
    )(degp, acc, zp, b2.reshape(1, -1), g2.reshape(1, -1),
      be2.reshape(1, -1), Wl1, bl1.reshape(1, -1), Wl2, bl2.reshape(1, -1),
      Wl3, bl3.reshape(1, -1))


def _tc_score(fx, fy):
    BI, BJ = 1024, 2048
    K = fx.shape[1]

    def body(fx_ref, fy_ref, o_ref):
        o_ref[...] = lax.dot_general(
            fx_ref[...], fy_ref[...], (((1,), (1,)), ((), ())),
            preferred_element_type=jnp.float32)

    return pl.pallas_call(
        body,
        grid=(N // BI, N // BJ),
        in_specs=[
            pl.BlockSpec((BI, K), lambda i, j: (i, 0)),
            pl.BlockSpec((BJ, K), lambda i, j: (j, 0)),
        ],
        out_specs=pl.BlockSpec((BI, BJ), lambda i, j: (i, j)),
        out_shape=jax.ShapeDtypeStruct((N, N), jnp.float32),
    )(fx, fy)


def kernel(x_m, x_d, mm_data, dd_data, mm_edge_index, dd_edge_index,
           W_gx1, b_gx1, g_nx1, be_nx1, W_gx2, b_gx2, g_nx2, be_nx2,
           W_gy1, b_gy1, g_ny1, be_ny1, W_gy2, b_gy2, g_ny2, be_ny2,
           W_lx1, b_lx1, W_lx2, b_lx2, W_lx3, b_lx3,
           W_ly1, b_ly1, W_ly2, b_ly2, W_ly3, b_ly3):
    s_m = mm_edge_index[0].astype(jnp.int32)
    t_m = mm_edge_index[1].astype(jnp.int32)
    s_d = dd_edge_index[0].astype(jnp.int32)
    t_d = dd_edge_index[1].astype(jnp.int32)
    flat_m = _tc_relayout(mm_data).reshape(-1)
    flat_d = _tc_relayout(dd_data).reshape(-1)
    ea_m, degp_m = _sc_prep(flat_m, s_m, t_m)
    ea_d, degp_d = _sc_prep(flat_d, s_d, t_d)
    xp_m = _tc_prescale(degp_m, x_m)
    xp_d = _tc_prescale(degp_d, x_d)
    acc1_m = _sc_agg(xp_m, s_m, t_m, ea_m)
    acc1_d = _sc_agg(xp_d, s_d, t_d, ea_d)
    zp_m = _tc_mid(degp_m, acc1_m, xp_m, W_gx1, b_gx1, g_nx1, be_nx1, W_gx2)
    zp_d = _tc_mid(degp_d, acc1_d, xp_d, W_gy1, b_gy1, g_ny1, be_ny1, W_gy2)
    acc2_m = _sc_agg(zp_m, s_m, t_m, ea_m)
    acc2_d = _sc_agg(zp_d, s_d, t_d, ea_d)
    fx = _tc_post(degp_m, acc2_m, zp_m, b_gx2, g_nx2, be_nx2,
                  W_lx1, b_lx1, W_lx2, b_lx2, W_lx3, b_lx3)
    fy = _tc_post(degp_d, acc2_d, zp_d, b_gy2, g_ny2, be_ny2,
                  W_ly1, b_ly1, W_ly2, b_ly2, W_ly3, b_ly3)
    return _tc_score(fx, fy)


# speedup vs baseline: 24.0733x; 1.4139x over previous
import functools

import jax
import jax.numpy as jnp
from jax import lax
from jax.experimental import pallas as pl
from jax.experimental.pallas import tpu as pltpu
from jax.experimental.pallas import tpu_sc as plsc

N = 8192
E = 262144
F = 128
NC, NS = 2, 16
NW = NC * NS
EPW = E // NW
RPS = N // NS

_MESH = dict(core_axis_name="c", subcore_axis_name="s")


def _sc_prep(data_flat, s_idx, t_idx):
    CH = 128
    NCHUNK = EPW // CH

    @functools.partial(
        pl.kernel,
        mesh=plsc.VectorSubcoreMesh(**_MESH),
        out_type=[
            jax.ShapeDtypeStruct((E,), jnp.float32),
            jax.ShapeDtypeStruct((NC, N), jnp.float32),
        ],
        scratch_types=[
            pltpu.VMEM((EPW,), jnp.int32),
            pltpu.VMEM((EPW,), jnp.int32),
            pltpu.VMEM((CH,), jnp.int32),
            pltpu.VMEM((CH,), jnp.int32),
            pltpu.VMEM((CH,), jnp.int32),
            pltpu.VMEM((CH,), jnp.int32),
            pltpu.VMEM((CH,), jnp.float32),
            pltpu.VMEM((CH,), jnp.float32),
            pltpu.VMEM((RPS,), jnp.float32),
            pltpu.VMEM_SHARED((N,), jnp.float32),
            pltpu.SemaphoreType.DMA,
            pltpu.SemaphoreType.DMA,
            pltpu.SemaphoreType.DMA,
            pltpu.SemaphoreType.DMA,
            pltpu.SemaphoreType.DMA,
            pltpu.SemaphoreType.DMA,
        ],
    )
    def kfn(data_hbm, s_hbm, t_hbm, ea_out, deg_out,
            s_v, t_v, gidx0, gidx1, tidx0, tidx1, eab0, eab1, zb, deg_sh,
            gsem0, gsem1, esem0, esem1, dsem0, dsem1):
        cid = lax.axis_index("c")
        sid = lax.axis_index("s")
        wid = cid * NS + sid
        base = wid * EPW
        slots = ((gidx0, tidx0, eab0, gsem0, esem0, dsem0),
                 (gidx1, tidx1, eab1, gsem1, esem1, dsem1))
        pltpu.sync_copy(s_hbm.at[pl.ds(base, EPW)], s_v)
        pltpu.sync_copy(t_hbm.at[pl.ds(base, EPW)], t_v)

        z16 = jnp.zeros((16,), jnp.float32)

        def zloop(i, carry):
            zb[pl.ds(i * 16, 16)] = z16
            return carry

        lax.fori_loop(0, RPS // 16, zloop, 0)
        pltpu.sync_copy(zb, deg_sh.at[pl.ds(sid * RPS, RPS)])
        plsc.subcore_barrier()

        def build(k, slot):
            gidx, tidx = slot[0], slot[1]
            cb = k * CH
            for g in range(CH // 16):
                s16 = s_v[pl.ds(cb + g * 16, 16)]
                t16 = t_v[pl.ds(cb + g * 16, 16)]
                gidx[pl.ds(g * 16, 16)] = (
                    ((s16 >> 3) << 16) + ((t16 >> 7) << 10)
                    + ((s16 & 7) << 7) + (t16 & 127))
                tidx[pl.ds(g * 16, 16)] = t16

        def fire_gather(slot):
            pltpu.async_copy(data_hbm.at[slot[0]], slot[2], slot[3])

        def wait_gather(slot):
            pltpu.make_async_copy(data_hbm.at[slot[0]], slot[2],
                                  slot[3]).wait()

        def fire_stores(k, slot):
            pltpu.async_copy(slot[2], ea_out.at[pl.ds(base + k * CH, CH)],
                             slot[4])
            pltpu.async_copy(slot[2], deg_sh.at[slot[1]], slot[5], add=True)

        def wait_stores(k, slot):
            pltpu.make_async_copy(slot[2], ea_out.at[pl.ds(base + k * CH, CH)],
                                  slot[4]).wait()
            pltpu.make_async_copy(slot[2], deg_sh.at[slot[1]], slot[5]).wait()

        build(0, slots[0])
        fire_gather(slots[0])

        def body(ci, carry):
            for b in range(2):
                k = ci * 2 + b
                cur, nxt = slots[b], slots[1 - b]

                @pl.when(k + 1 < NCHUNK)
                def _():
                    @pl.when(k > 0)
                    def _():
                        wait_stores(k - 1, nxt)
                    build(k + 1, nxt)
                    fire_gather(nxt)

                wait_gather(cur)
                fire_stores(k, cur)
            return carry

        lax.fori_loop(0, NCHUNK // 2, body, 0)
        wait_stores(NCHUNK - 2, slots[0])
        wait_stores(NCHUNK - 1, slots[1])
        plsc.subcore_barrier()
        pltpu.sync_copy(deg_sh.at[pl.ds(sid * RPS, RPS)],
                        deg_out.at[cid, pl.ds(sid * RPS, RPS)])

    return kfn(data_flat, s_idx, t_idx)


def _sc_agg(xp, s_idx, t_idx, ea):
    CH = 64
    NCHUNK = EPW // CH
    ZR = 64

    @functools.partial(
        pl.kernel,
        mesh=plsc.VectorSubcoreMesh(**_MESH),
        out_type=jax.ShapeDtypeStruct((NC, N, F), jnp.float32),
        scratch_types=[
            pltpu.VMEM((EPW,), jnp.int32),
            pltpu.VMEM((EPW,), jnp.int32),
            pltpu.VMEM((EPW,), jnp.float32),
            pltpu.VMEM((CH,), jnp.int32),
            pltpu.VMEM((CH,), jnp.int32),
            pltpu.VMEM((CH,), jnp.int32),
            pltpu.VMEM((CH,), jnp.int32),
            pltpu.VMEM((CH,), jnp.float32),
            pltpu.VMEM((CH,), jnp.float32),
            pltpu.VMEM((CH, F), jnp.float32),
            pltpu.VMEM((CH, F), jnp.float32),
            pltpu.VMEM((ZR, F), jnp.float32),
            pltpu.VMEM_SHARED((N, F), jnp.float32),
            pltpu.SemaphoreType.DMA,
            pltpu.SemaphoreType.DMA,
            pltpu.SemaphoreType.DMA,
            pltpu.SemaphoreType.DMA,
        ],
    )
    def kfn(xp_hbm, s_hbm, t_hbm, ea_hbm, acc_out,
            s_v, t_v, ea_v, gidx0, gidx1, tidx0, tidx1, eab0, eab1,
            rows0, rows1, zb, acc_sh, gsem0, gsem1, ssem0, ssem1):
        cid = lax.axis_index("c")
        sid = lax.axis_index("s")
        wid = cid * NS + sid
        base = wid * EPW
        slots = ((gidx0, tidx0, eab0, rows0, gsem0, ssem0),
                 (gidx1, tidx1, eab1, rows1, gsem1, ssem1))
        pltpu.sync_copy(s_hbm.at[pl.ds(base, EPW)], s_v)
        pltpu.sync_copy(t_hbm.at[pl.ds(base, EPW)], t_v)
        pltpu.sync_copy(ea_hbm.at[pl.ds(base, EPW)], ea_v)

        z16 = jnp.zeros((16,), jnp.float32)

        def zloop(i, carry):
            r = i // (F // 16)
            q = i % (F // 16)
            zb[r, pl.ds(q * 16, 16)] = z16
            return carry

        lax.fori_loop(0, ZR * (F // 16), zloop, 0)
        for q in range(RPS // ZR):
            pltpu.sync_copy(zb, acc_sh.at[pl.ds(sid * RPS + q * ZR, ZR), :])
        plsc.subcore_barrier()

        def build(k, slot):
            gidx, tidx, eab = slot[0], slot[1], slot[2]
            cb = k * CH
            for g in range(CH // 16):
                s16 = s_v[pl.ds(cb + g * 16, 16)]
                t16 = t_v[pl.ds(cb + g * 16, 16)]
                e16 = ea_v[pl.ds(cb + g * 16, 16)]
                gidx[pl.ds(g * 16, 16)] = s16
                tidx[pl.ds(g * 16, 16)] = t16
                eab[pl.ds(g * 16, 16)] = e16

        def fire_gather(slot):
            pltpu.async_copy(xp_hbm.at[slot[0]], slot[3], slot[4])

        def wait_gather(slot):
            pltpu.make_async_copy(xp_hbm.at[slot[0]], slot[3], slot[4]).wait()

        def scale(slot):
            eab, rows = slot[2], slot[3]
            for g in range(CH // 16):
                av = eab[pl.ds(g * 16, 16)]
                for l in range(16):
                    e = g * 16 + l
                    a = av[l]
                    for j in range(F // 16):
                        rows[e, pl.ds(j * 16, 16)] = (
                            rows[e, pl.ds(j * 16, 16)] * a)

        def fire_scatter(slot):
            pltpu.async_copy(slot[3], acc_sh.at[slot[1]], slot[5], add=True)

        def wait_scatter(slot):
            pltpu.make_async_copy(slot[3], acc_sh.at[slot[1]], slot[5]).wait()

        build(0, slots[0])
        fire_gather(slots[0])

        def body(ci, carry):
            for b in range(2):
                k = ci * 2 + b
                cur, nxt = slots[b], slots[1 - b]

                @pl.when(k + 1 < NCHUNK)
                def _():
                    @pl.when(k > 0)
                    def _():
                        wait_scatter(nxt)
                    build(k + 1, nxt)
                    fire_gather(nxt)

                wait_gather(cur)
                scale(cur)
                fire_scatter(cur)
            return carry

        lax.fori_loop(0, NCHUNK // 2, body, 0)
        wait_scatter(slots[0])
        wait_scatter(slots[1])
        plsc.subcore_barrier()
        for q in range(RPS // ZR):
            rb = sid * RPS + q * ZR
            pltpu.sync_copy(acc_sh.at[pl.ds(rb, ZR), :],
                            acc_out.at[cid, pl.ds(rb, ZR), :])

    return kfn(xp, s_idx, t_idx, ea)


_BR = 1024


def _tiled_view(data):
    return data.reshape(N // 8, 8, N // 128, 128).transpose(
        0, 2, 1, 3).reshape(-1)


def _dis_block(degp):
    deg = degp[0] + degp[1] + 1.0
    return jnp.where(deg > 0, lax.rsqrt(deg), 0.0)


def _tc_prescale(degp, x):
    def body(degp_ref, x_ref, o_ref):
        dis = _dis_block(degp_ref[...])
        o_ref[...] = x_ref[...] * dis[:, None]

    return pl.pallas_call(
        body,
        grid=(N // _BR,),
        in_specs=[
            pl.BlockSpec((NC, _BR), lambda i: (0, i)),
            pl.BlockSpec((_BR, F), lambda i: (i, 0)),
        ],
        out_specs=pl.BlockSpec((_BR, F), lambda i: (i, 0)),
        out_shape=jax.ShapeDtypeStruct((N, F), jnp.float32),
    )(degp, x)


def _ln(x, g, b):
    mu = jnp.mean(x, axis=-1, keepdims=True)
    var = jnp.mean((x - mu) ** 2, axis=-1, keepdims=True)
    return (x - mu) / jnp.sqrt(var + 1e-5) * g + b


def _tc_mid(degp, acc, xp, W1, b1, g1, be1, W2):
    def body(degp_ref, a_ref, xp_ref, w1_ref, b1_ref, g1_ref, be1_ref,
             w2_ref, o_ref):
        dis = _dis_block(degp_ref[...])[:, None]
        pre = dis * (a_ref[0] + a_ref[1] + xp_ref[...])
        h = jnp.dot(pre, w1_ref[...], preferred_element_type=jnp.float32)
        h = jnp.maximum(h + b1_ref[0], 0.0)
        h = _ln(h, g1_ref[0], be1_ref[0])
        z = jnp.dot(h, w2_ref[...], preferred_element_type=jnp.float32)
        o_ref[...] = dis * z

    C2 = W1.shape[1]
    return pl.pallas_call(
        body,
        grid=(N // _BR,),
        in_specs=[
            pl.BlockSpec((NC, _BR), lambda i: (0, i)),
            pl.BlockSpec((NC, _BR, F), lambda i: (0, i, 0)),
            pl.BlockSpec((_BR, F), lambda i: (i, 0)),
            pl.BlockSpec((F, C2), lambda i: (0, 0)),
            pl.BlockSpec((1, C2), lambda i: (0, 0)),
            pl.BlockSpec((1, C2), lambda i: (0, 0)),
            pl.BlockSpec((1, C2), lambda i: (0, 0)),
            pl.BlockSpec((C2, F), lambda i: (0, 0)),
        ],
        out_specs=pl.BlockSpec((_BR, F), lambda i: (i, 0)),
        out_shape=jax.ShapeDtypeStruct((N, F), jnp.float32),
    )(degp, acc, xp, W1, b1.reshape(1, -1), g1.reshape(1, -1),
      be1.reshape(1, -1), W2)


def _tc_post(degp, acc, zp, b2, g2, be2, Wl1, bl1, Wl2, bl2, Wl3, bl3):
    K = Wl3.shape[1]

    def body(degp_ref, a_ref, zp_ref, b2_ref, g2_ref, be2_ref,
             w1_ref, c1_ref, w2_ref, c2_ref, w3_ref, c3_ref, o_ref):
        dis = _dis_block(degp_ref[...])[:, None]
        x = dis * (a_ref[0] + a_ref[1] + zp_ref[...])
        x = jnp.maximum(x + b2_ref[0], 0.0)
        x = _ln(x, g2_ref[0], be2_ref[0])
        h = jnp.dot(x, w1_ref[...], preferred_element_type=jnp.float32)
        h = jnp.maximum(h + c1_ref[0], 0.0)
        h = jnp.dot(h, w2_ref[...], preferred_element_type=jnp.float32)
        h = jnp.maximum(h + c2_ref[0], 0.0)
        h = jnp.dot(h, w3_ref[...], preferred_element_type=jnp.float32)
        o_ref[...] = jnp.maximum(h + c3_ref[0], 0.0)

    H1, H2 = Wl1.shape[1], Wl2.shape[1]
    return pl.pallas_call(
        body,
        grid=(N // _BR,),
        in_specs=[
            pl.BlockSpec((NC, _BR), lambda i: (0, i)),
            pl.BlockSpec((NC, _BR, F), lambda i: (0, i, 0)),
            pl.BlockSpec((_BR, F), lambda i: (i, 0)),
            pl.BlockSpec((1, F), lambda i: (0, 0)),
            pl.BlockSpec((1, F), lambda i: (0, 0)),
            pl.BlockSpec((1, F), lambda i: (0, 0)),
            pl.BlockSpec((F, H1), lambda i: (0, 0)),
            pl.BlockSpec((1, H1), lambda i: (0, 0)),
            pl.BlockSpec((H1, H2), lambda i: (0, 0)),
            pl.BlockSpec((1, H2), lambda i: (0, 0)),
            pl.BlockSpec((H2, K), lambda i: (0, 0)),
            pl.BlockSpec((1, K), lambda i: (0, 0)),
        ],
        out_specs=pl.BlockSpec((_BR, K), lambda i: (i, 0)),
        out_shape=jax.ShapeDtypeStruct((N, K), jnp.float32),
    )(degp, acc, zp, b2.reshape(1, -1), g2.reshape(1, -1),
      be2.reshape(1, -1), Wl1, bl1.reshape(1, -1), Wl2, bl2.reshape(1, -1),
      Wl3, bl3.reshape(1, -1))


def _tc_score(fx, fy):
    BI, BJ = 1024, 2048
    K = fx.shape[1]

    def body(fx_ref, fy_ref, o_ref):
        o_ref[...] = lax.dot_general(
            fx_ref[...], fy_ref[...], (((1,), (1,)), ((), ())),
            preferred_element_type=jnp.float32)

    return pl.pallas_call(
        body,
        grid=(N // BI, N // BJ),
        in_specs=[
            pl.BlockSpec((BI, K), lambda i, j: (i, 0)),
            pl.BlockSpec((BJ, K), lambda i, j: (j, 0)),
        ],
        out_specs=pl.BlockSpec((BI, BJ), lambda i, j: (i, j)),
        out_shape=jax.ShapeDtypeStruct((N, N), jnp.float32),
    )(fx, fy)


def kernel(x_m, x_d, mm_data, dd_data, mm_edge_index, dd_edge_index,
           W_gx1, b_gx1, g_nx1, be_nx1, W_gx2, b_gx2, g_nx2, be_nx2,
           W_gy1, b_gy1, g_ny1, be_ny1, W_gy2, b_gy2, g_ny2, be_ny2,
           W_lx1, b_lx1, W_lx2, b_lx2, W_lx3, b_lx3,
           W_ly1, b_ly1, W_ly2, b_ly2, W_ly3, b_ly3):
    s_m = mm_edge_index[0].astype(jnp.int32)
    t_m = mm_edge_index[1].astype(jnp.int32)
    s_d = dd_edge_index[0].astype(jnp.int32)
    t_d = dd_edge_index[1].astype(jnp.int32)
    flat_m = _tiled_view(mm_data)
    flat_d = _tiled_view(dd_data)
    ea_m, degp_m = _sc_prep(flat_m, s_m, t_m)
    ea_d, degp_d = _sc_prep(flat_d, s_d, t_d)
    xp_m = _tc_prescale(degp_m, x_m)
    xp_d = _tc_prescale(degp_d, x_d)
    acc1_m = _sc_agg(xp_m, s_m, t_m, ea_m)
    acc1_d = _sc_agg(xp_d, s_d, t_d, ea_d)
    zp_m = _tc_mid(degp_m, acc1_m, xp_m, W_gx1, b_gx1, g_nx1, be_nx1, W_gx2)
    zp_d = _tc_mid(degp_d, acc1_d, xp_d, W_gy1, b_gy1, g_ny1, be_ny1, W_gy2)
    acc2_m = _sc_agg(zp_m, s_m, t_m, ea_m)
    acc2_d = _sc_agg(zp_d, s_d, t_d, ea_d)
    fx = _tc_post(degp_m, acc2_m, zp_m, b_gx2, g_nx2, be_nx2,
                  W_lx1, b_lx1, W_lx2, b_lx2, W_lx3, b_lx3)
    fy = _tc_post(degp_d, acc2_d, zp_d, b_gy2, g_ny2, be_ny2,
                  W_ly1, b_ly1, W_ly2, b_ly2, W_ly3, b_ly3)
    return _tc_score(fx, fy)
